# Initial kernel scaffold; baseline (speedup 1.0000x reference)
import jax
import jax.numpy as jnp
from jax.experimental import pallas as pl


def kernel(x, w1, b1, w2, b2):
    raise NotImplementedError("write your pallas kernel here")


# speedup vs baseline: 1.0201x; 1.0201x over previous
import jax
import jax.numpy as jnp
from jax.experimental import pallas as pl
from jax.experimental.pallas import tpu as pltpu

_BATCH_TILE = 8192


def _round_up(n: int, m: int) -> int:
    return pl.cdiv(n, m) * m


def _mlp_kernel(x_ref, w1_ref, b1_ref, w2_ref, b2_ref, o_ref, *, out: int):
    x = x_ref[...].astype(jnp.bfloat16)
    h = jnp.dot(x, w1_ref[...], preferred_element_type=jnp.float32)
    h = jnp.maximum(h + b1_ref[...], 0.0)
    y = jnp.dot(h.astype(jnp.bfloat16), w2_ref[...],
                preferred_element_type=jnp.float32)
    y = y + b2_ref[...]
    o_ref[...] = y[:, :out]


def kernel(x, w1, b1, w2, b2):
    B, In = x.shape
    H = w1.shape[1]
    Out = w2.shape[1]

    H_p = _round_up(H, 128)
    Out_p = _round_up(Out, 128)

    w1p = jnp.zeros((In, H_p), jnp.float32).at[:, :H].set(w1)
    b1p = jnp.zeros((1, H_p), jnp.float32).at[:, :H].set(b1)
    w2p = jnp.zeros((H_p, Out_p), jnp.float32).at[:H, :Out].set(w2)
    b2p = jnp.zeros((1, Out_p), jnp.float32).at[:, :Out].set(b2)

    w1p = w1p.astype(jnp.bfloat16)
    w2p = w2p.astype(jnp.bfloat16)

    n_tiles = max(2, pl.cdiv(B, _BATCH_TILE))
    n_tiles += n_tiles % 2
    tb = _round_up(pl.cdiv(B, n_tiles), 8)
    grid = (pl.cdiv(B, tb),)

    import functools
    body = functools.partial(_mlp_kernel, out=Out)

    y = pl.pallas_call(
        body,
        out_shape=jax.ShapeDtypeStruct((B, Out), jnp.float32),
        grid=grid,
        in_specs=[
            pl.BlockSpec((tb, In), lambda i: (i, 0)),
            pl.BlockSpec((In, H_p), lambda i: (0, 0)),
            pl.BlockSpec((1, H_p), lambda i: (0, 0)),
            pl.BlockSpec((H_p, Out_p), lambda i: (0, 0)),
            pl.BlockSpec((1, Out_p), lambda i: (0, 0)),
        ],
        out_specs=pl.BlockSpec((tb, Out), lambda i: (i, 0)),
        compiler_params=pltpu.CompilerParams(
            dimension_semantics=("parallel",),
        ),
    )(x, w1p, b1p, w2p, b2p)
    return y


# speedup vs baseline: 4.3727x; 4.2867x over previous
import functools

import jax
import jax.numpy as jnp
from jax.experimental import pallas as pl
from jax.experimental.pallas import tpu as pltpu

_LANE_TILE = 8192


def _round_up(n: int, m: int) -> int:
    return pl.cdiv(n, m) * m


def _mlp_kernel(xt_ref, w1t_ref, b1t_ref, w2t_ref, b2t_ref, o_ref, *, out: int):
    xt = xt_ref[...].astype(jnp.bfloat16)
    h = jax.lax.dot_general(w1t_ref[...], xt,
                            (((1,), (0,)), ((), ())),
                            preferred_element_type=jnp.float32)
    h = jnp.maximum(h + b1t_ref[...], 0.0)
    y = jax.lax.dot_general(w2t_ref[...], h.astype(jnp.bfloat16),
                            (((1,), (0,)), ((), ())),
                            preferred_element_type=jnp.float32)
    y = y + b2t_ref[...]
    o_ref[...] = y[:out, :]


def kernel(x, w1, b1, w2, b2):
    B, In = x.shape
    H = w1.shape[1]
    Out = w2.shape[1]

    H_p = _round_up(H, 128)
    Out_p = _round_up(Out, 128)

    w1t = jnp.zeros((H_p, In), jnp.float32).at[:H, :].set(w1.T)
    b1t = jnp.zeros((H_p, 1), jnp.float32).at[:H, :].set(b1.T)
    w2t = jnp.zeros((Out_p, H_p), jnp.float32).at[:Out, :H].set(w2.T)
    b2t = jnp.zeros((Out_p, 1), jnp.float32).at[:Out, :].set(b2.T)
    w1t = w1t.astype(jnp.bfloat16)
    w2t = w2t.astype(jnp.bfloat16)

    xt = x.T

    n_tiles = max(2, pl.cdiv(B, _LANE_TILE))
    n_tiles += n_tiles % 2
    tb = _round_up(pl.cdiv(B, n_tiles), 128)
    grid = (pl.cdiv(B, tb),)

    body = functools.partial(_mlp_kernel, out=Out)

    yt = pl.pallas_call(
        body,
        out_shape=jax.ShapeDtypeStruct((Out, B), jnp.float32),
        grid=grid,
        in_specs=[
            pl.BlockSpec((In, tb), lambda i: (0, i)),
            pl.BlockSpec((H_p, In), lambda i: (0, 0)),
            pl.BlockSpec((H_p, 1), lambda i: (0, 0)),
            pl.BlockSpec((Out_p, H_p), lambda i: (0, 0)),
            pl.BlockSpec((Out_p, 1), lambda i: (0, 0)),
        ],
        out_specs=pl.BlockSpec((Out, tb), lambda i: (0, i)),
        compiler_params=pltpu.CompilerParams(
            dimension_semantics=("parallel",),
        ),
    )(xt, w1t, b1t, w2t, b2t)
    return yt.T


# speedup vs baseline: 5.0580x; 1.1567x over previous
import functools

import jax
import jax.numpy as jnp
from jax.experimental import pallas as pl
from jax.experimental.pallas import tpu as pltpu

_LANE_TILE = 16384


def _round_up(n: int, m: int) -> int:
    return pl.cdiv(n, m) * m


def _mlp_kernel(xt_ref, w1t_ref, b1t_ref, w2t_ref, b2t_ref, o_ref, *, out: int):
    xt = xt_ref[...].astype(jnp.bfloat16)
    h = jax.lax.dot_general(w1t_ref[...], xt,
                            (((1,), (0,)), ((), ())),
                            preferred_element_type=jnp.float32)
    h = jnp.maximum(h + b1t_ref[...], 0.0)
    y = jax.lax.dot_general(w2t_ref[...], h.astype(jnp.bfloat16),
                            (((1,), (0,)), ((), ())),
                            preferred_element_type=jnp.float32)
    y = y + b2t_ref[...]
    o_ref[...] = y[:out, :]


def kernel(x, w1, b1, w2, b2):
    B, In = x.shape
    H = w1.shape[1]
    Out = w2.shape[1]

    H_p = _round_up(H, 128)
    Out_p = _round_up(Out, 128)

    w1t = jnp.zeros((H_p, In), jnp.float32).at[:H, :].set(w1.T)
    b1t = jnp.zeros((H_p, 1), jnp.float32).at[:H, :].set(b1.T)
    w2t = jnp.zeros((Out_p, H_p), jnp.float32).at[:Out, :H].set(w2.T)
    b2t = jnp.zeros((Out_p, 1), jnp.float32).at[:Out, :].set(b2.T)
    w1t = w1t.astype(jnp.bfloat16)
    w2t = w2t.astype(jnp.bfloat16)

    xt = x.T

    n_tiles = max(2, pl.cdiv(B, _LANE_TILE))
    n_tiles += n_tiles % 2
    tb = _round_up(pl.cdiv(B, n_tiles), 128)
    grid = (pl.cdiv(B, tb),)

    body = functools.partial(_mlp_kernel, out=Out)

    yt = pl.pallas_call(
        body,
        out_shape=jax.ShapeDtypeStruct((Out, B), jnp.float32),
        grid=grid,
        in_specs=[
            pl.BlockSpec((In, tb), lambda i: (0, i)),
            pl.BlockSpec((H_p, In), lambda i: (0, 0)),
            pl.BlockSpec((H_p, 1), lambda i: (0, 0)),
            pl.BlockSpec((Out_p, H_p), lambda i: (0, 0)),
            pl.BlockSpec((Out_p, 1), lambda i: (0, 0)),
        ],
        out_specs=pl.BlockSpec((Out, tb), lambda i: (0, i)),
        compiler_params=pltpu.CompilerParams(
            dimension_semantics=("parallel",),
        ),
    )(xt, w1t, b1t, w2t, b2t)
    return yt.T


# speedup vs baseline: 5.3522x; 1.0582x over previous
import functools

import jax
import jax.numpy as jnp
from jax.experimental import pallas as pl
from jax.experimental.pallas import tpu as pltpu

_LANE_TILE = 32768


def _round_up(n: int, m: int) -> int:
    return pl.cdiv(n, m) * m


def _mlp_kernel(xt_ref, w1t_ref, b1t_ref, w2t_ref, b2t_ref, o_ref, *, out: int):
    xt = xt_ref[...].astype(jnp.bfloat16)
    h = jax.lax.dot_general(w1t_ref[...], xt,
                            (((1,), (0,)), ((), ())),
                            preferred_element_type=jnp.float32)
    h = jnp.maximum(h + b1t_ref[...], 0.0)
    y = jax.lax.dot_general(w2t_ref[...], h.astype(jnp.bfloat16),
                            (((1,), (0,)), ((), ())),
                            preferred_element_type=jnp.float32)
    y = y + b2t_ref[...]
    o_ref[...] = y[:out, :]


def kernel(x, w1, b1, w2, b2):
    B, In = x.shape
    H = w1.shape[1]
    Out = w2.shape[1]

    H_p = _round_up(H, 128)
    Out_p = _round_up(Out, 128)

    w1t = jnp.zeros((H_p, In), jnp.float32).at[:H, :].set(w1.T)
    b1t = jnp.zeros((H_p, 1), jnp.float32).at[:H, :].set(b1.T)
    w2t = jnp.zeros((Out_p, H_p), jnp.float32).at[:Out, :H].set(w2.T)
    b2t = jnp.zeros((Out_p, 1), jnp.float32).at[:Out, :].set(b2.T)
    w1t = w1t.astype(jnp.bfloat16)
    w2t = w2t.astype(jnp.bfloat16)

    xt = x.T

    n_tiles = max(2, pl.cdiv(B, _LANE_TILE))
    n_tiles += n_tiles % 2
    tb = _round_up(pl.cdiv(B, n_tiles), 128)
    grid = (pl.cdiv(B, tb),)

    body = functools.partial(_mlp_kernel, out=Out)

    yt = pl.pallas_call(
        body,
        out_shape=jax.ShapeDtypeStruct((Out, B), jnp.float32),
        grid=grid,
        in_specs=[
            pl.BlockSpec((In, tb), lambda i: (0, i)),
            pl.BlockSpec((H_p, In), lambda i: (0, 0)),
            pl.BlockSpec((H_p, 1), lambda i: (0, 0)),
            pl.BlockSpec((Out_p, H_p), lambda i: (0, 0)),
            pl.BlockSpec((Out_p, 1), lambda i: (0, 0)),
        ],
        out_specs=pl.BlockSpec((Out, tb), lambda i: (0, i)),
        compiler_params=pltpu.CompilerParams(
            dimension_semantics=("parallel",),
        ),
    )(xt, w1t, b1t, w2t, b2t)
    return yt.T


# speedup vs baseline: 6.1820x; 1.1550x over previous
import jax
import jax.numpy as jnp
from jax.experimental import pallas as pl
from jax.experimental.pallas import tpu as pltpu

_LANE_TILE = 32768

_TN_DIMS = (((0,), (0,)), ((), ()))


def _round_up(n: int, m: int) -> int:
    return pl.cdiv(n, m) * m


def _mlp_kernel(xt_ref, w1_ref, b1t_ref, w2_ref, b2t_ref, o_ref):
    xt = xt_ref[...].astype(jnp.bfloat16)
    h = jax.lax.dot_general(w1_ref[...].astype(jnp.bfloat16), xt, _TN_DIMS,
                            preferred_element_type=jnp.float32)
    h = jnp.maximum(h + b1t_ref[...], 0.0)
    y = jax.lax.dot_general(w2_ref[...].astype(jnp.bfloat16),
                            h.astype(jnp.bfloat16), _TN_DIMS,
                            preferred_element_type=jnp.float32)
    o_ref[...] = y + b2t_ref[...]


def kernel(x, w1, b1, w2, b2):
    B, In = x.shape
    H = w1.shape[1]
    Out = w2.shape[1]

    xt = x.T
    b1t = b1.T
    b2t = b2.T

    n_tiles = max(2, pl.cdiv(B, _LANE_TILE))
    n_tiles += n_tiles % 2
    tb = _round_up(pl.cdiv(B, n_tiles), 128)
    grid = (pl.cdiv(B, tb),)

    yt = pl.pallas_call(
        _mlp_kernel,
        out_shape=jax.ShapeDtypeStruct((Out, B), jnp.float32),
        grid=grid,
        in_specs=[
            pl.BlockSpec((In, tb), lambda i: (0, i)),
            pl.BlockSpec((In, H), lambda i: (0, 0)),
            pl.BlockSpec((H, 1), lambda i: (0, 0)),
            pl.BlockSpec((H, Out), lambda i: (0, 0)),
            pl.BlockSpec((Out, 1), lambda i: (0, 0)),
        ],
        out_specs=pl.BlockSpec((Out, tb), lambda i: (0, i)),
        compiler_params=pltpu.CompilerParams(
            dimension_semantics=("parallel",),
        ),
    )(xt, w1, b1t, w2, b2t)
    return yt.T


# speedup vs baseline: 6.7883x; 1.0981x over previous
import functools

import jax
import jax.numpy as jnp
from jax.experimental import pallas as pl
from jax.experimental.pallas import tpu as pltpu

_LANE_TILE = 32768


def _round_up(n: int, m: int) -> int:
    return pl.cdiv(n, m) * m


def _mlp_kernel(xt_ref, w1t_ref, w2t_ref, bt_ref, o_ref, *, hidden: int):
    xt = xt_ref[...].astype(jnp.bfloat16)
    h = jax.lax.dot_general(w1t_ref[...].astype(jnp.bfloat16), xt,
                            (((1,), (0,)), ((), ())),
                            preferred_element_type=jnp.float32)
    h = jnp.maximum(h + bt_ref[:hidden, :], 0.0)
    y = jax.lax.dot_general(w2t_ref[...].astype(jnp.bfloat16),
                            h.astype(jnp.bfloat16),
                            (((1,), (0,)), ((), ())),
                            preferred_element_type=jnp.float32)
    o_ref[...] = y + bt_ref[hidden:, :]


def kernel(x, w1, b1, w2, b2):
    B, In = x.shape
    H = w1.shape[1]
    Out = w2.shape[1]

    xt = x.T
    w1t = w1.T
    w2t = w2.T
    bt = jnp.concatenate([b1, b2], axis=1).T

    n_tiles = max(2, pl.cdiv(B, _LANE_TILE))
    n_tiles += n_tiles % 2
    tb = _round_up(pl.cdiv(B, n_tiles), 128)
    grid = (pl.cdiv(B, tb),)

    body = functools.partial(_mlp_kernel, hidden=H)

    yt = pl.pallas_call(
        body,
        out_shape=jax.ShapeDtypeStruct((Out, B), jnp.float32),
        grid=grid,
        in_specs=[
            pl.BlockSpec((In, tb), lambda i: (0, i)),
            pl.BlockSpec((H, In), lambda i: (0, 0)),
            pl.BlockSpec((Out, H), lambda i: (0, 0)),
            pl.BlockSpec((H + Out, 1), lambda i: (0, 0)),
        ],
        out_specs=pl.BlockSpec((Out, tb), lambda i: (0, i)),
        compiler_params=pltpu.CompilerParams(
            dimension_semantics=("parallel",),
        ),
    )(xt, w1t, w2t, bt)
    return yt.T


# speedup vs baseline: 7.0160x; 1.0335x over previous
import functools

import jax
import jax.numpy as jnp
from jax.experimental import pallas as pl
from jax.experimental.pallas import tpu as pltpu

_LANE_TILE = 32768


def _round_up(n: int, m: int) -> int:
    return pl.cdiv(n, m) * m


def _mlp_kernel(xt_ref, w1t_ref, w2t_ref, b1_ref, b2_ref, o_ref):
    xt = xt_ref[...].astype(jnp.bfloat16)
    h = jax.lax.dot_general(w1t_ref[...].astype(jnp.bfloat16), xt,
                            (((1,), (0,)), ((), ())),
                            preferred_element_type=jnp.float32)
    h = jnp.maximum(h + jnp.transpose(b1_ref[...]), 0.0)
    y = jax.lax.dot_general(w2t_ref[...].astype(jnp.bfloat16),
                            h.astype(jnp.bfloat16),
                            (((1,), (0,)), ((), ())),
                            preferred_element_type=jnp.float32)
    o_ref[...] = y + jnp.transpose(b2_ref[...])


def kernel(x, w1, b1, w2, b2):
    B, In = x.shape
    H = w1.shape[1]
    Out = w2.shape[1]

    xt = x.T
    w1t = w1.T
    w2t = w2.T

    n_tiles = max(2, pl.cdiv(B, _LANE_TILE))
    n_tiles += n_tiles % 2
    tb = _round_up(pl.cdiv(B, n_tiles), 128)
    grid = (pl.cdiv(B, tb),)

    yt = pl.pallas_call(
        _mlp_kernel,
        out_shape=jax.ShapeDtypeStruct((Out, B), jnp.float32),
        grid=grid,
        in_specs=[
            pl.BlockSpec((In, tb), lambda i: (0, i)),
            pl.BlockSpec((H, In), lambda i: (0, 0)),
            pl.BlockSpec((Out, H), lambda i: (0, 0)),
            pl.BlockSpec((1, H), lambda i: (0, 0)),
            pl.BlockSpec((1, Out), lambda i: (0, 0)),
        ],
        out_specs=pl.BlockSpec((Out, tb), lambda i: (0, i)),
        compiler_params=pltpu.CompilerParams(
            dimension_semantics=("parallel",),
        ),
    )(xt, w1t, w2t, b1, b2)
    return yt.T
